# baseline (speedup 1.0000x reference)
import functools

import jax
import jax.numpy as jnp
from jax import lax
from jax.experimental import pallas as pl
from jax.experimental.pallas import tpu as pltpu
from jax.experimental.pallas import tpu_sc as plsc

_BS1 = 12
_BS2 = 16
_SC_B = 64
_ROWS_PER_SAMPLE = 64
_LANES = 2048
_CH = 16


def _pass1_body(n_opes, n_mas,
                proc_ref, opes_ref, mas_ref,
                part_out, opes_out, mas_out):
    @pl.when(pl.program_id(0) == 0)
    def _mas():
        y = mas_ref[...]
        my = jnp.mean(y, axis=0, keepdims=True)
        dy = y - my
        vy = jnp.sum(dy * dy, axis=0, keepdims=True) * (1.0 / (n_mas - 1.0))
        mas_out[...] = dy / (jnp.sqrt(vy) + 1e-5)

    x = proc_ref[...] - 0.5
    ps = jnp.sum(x, axis=(0, 1))
    ps2 = jnp.sum(x * x, axis=(0, 1))
    part_out[...] = jnp.stack([ps, ps2]).reshape(1, 2, -1)

    z = opes_ref[...]
    m = jnp.mean(z, axis=2, keepdims=True)
    d = z - m
    v = jnp.sum(d * d, axis=2, keepdims=True) * (1.0 / (n_opes - 1.0))
    opes_out[...] = d / (jnp.sqrt(v) + 1e-5)


def _sc_reduce_body(rows_per_worker, start_row, x_hbm, out_hbm, buf, obuf):
    wid = lax.axis_index("s") * 2 + lax.axis_index("c")
    r0 = start_row + wid * rows_per_worker
    n_chunks = rows_per_worker // _CH
    vregs = _CH * _LANES // 16

    acc = jnp.zeros((16,), jnp.float32)
    acc2 = jnp.zeros((16,), jnp.float32)
    for c in range(n_chunks):
        pltpu.sync_copy(x_hbm.at[pl.ds(r0 + c * _CH, _CH)], buf)

        def _step(k, carry):
            a, a2 = carry
            v = buf[k // 128, pl.ds((k % 128) * 16, 16)] - 0.5
            return a + v, a2 + v * v

        acc, acc2 = lax.fori_loop(0, vregs, _step, (acc, acc2))

    obuf[0, :] = acc
    obuf[1, :] = acc2
    pltpu.sync_copy(obuf, out_hbm.at[wid])


def _pass2_body(n_total, proc_ref, part_ref, scp_ref, proc_out):
    parts = part_ref[...]
    scp = scp_ref[...]
    s = jnp.sum(parts[:, 0:1, :]) + jnp.sum(scp[:, 0:1, :])
    s2 = jnp.sum(parts[:, 1:2, :]) + jnp.sum(scp[:, 1:2, :])
    n = float(n_total)
    gvar = (s2 - s * s / n) / (n - 1.0)
    ginv = 1.0 / (jnp.sqrt(gvar) + 1e-5)
    gmean = s / n
    proc_out[...] = ((proc_ref[...] - 0.5) - gmean) * ginv


def kernel(raw_opes, raw_mas, proc_time, batch_idxes, nums_opes):
    B, N_OPES, D_OPE = raw_opes.shape
    _, N_MAS, D_MA = raw_mas.shape
    n_total = B * N_OPES * N_MAS
    tc_b = B - _SC_B
    G1 = tc_b // _BS1
    G2 = B // _BS2
    bs_o = B // G1

    pt = jnp.transpose(proc_time, (0, 2, 1))
    ot = jnp.transpose(raw_opes, (0, 2, 1))
    mt = jnp.transpose(raw_mas, (1, 2, 0))
    pt2d = pt.reshape(B * N_MAS, N_OPES)

    rows_per_worker = _SC_B * _ROWS_PER_SAMPLE // 32
    start_row = tc_b * _ROWS_PER_SAMPLE

    sc_parts = pl.kernel(
        functools.partial(_sc_reduce_body, rows_per_worker, start_row),
        mesh=plsc.VectorSubcoreMesh(core_axis_name="c", subcore_axis_name="s"),
        out_type=jax.ShapeDtypeStruct((32, 2, 16), jnp.float32),
        scratch_types=[
            pltpu.VMEM((_CH, _LANES), jnp.float32),
            pltpu.VMEM((2, 16), jnp.float32),
        ],
        compiler_params=pltpu.CompilerParams(use_tc_tiling_on_sc=True),
    )(pt2d)

    parts, on, mn = pl.pallas_call(
        functools.partial(_pass1_body, N_OPES, N_MAS),
        grid=(G1,),
        in_specs=[
            pl.BlockSpec((_BS1, N_MAS, N_OPES), lambda i: (i, 0, 0)),
            pl.BlockSpec((bs_o, D_OPE, N_OPES), lambda i: (i, 0, 0)),
            pl.BlockSpec((N_MAS, D_MA, B), lambda i: (0, 0, 0)),
        ],
        out_specs=[
            pl.BlockSpec((1, 2, N_OPES), lambda i: (i, 0, 0)),
            pl.BlockSpec((bs_o, D_OPE, N_OPES), lambda i: (i, 0, 0)),
            pl.BlockSpec((N_MAS, D_MA, B), lambda i: (0, 0, 0)),
        ],
        out_shape=[
            jax.ShapeDtypeStruct((G1, 2, N_OPES), jnp.float32),
            jax.ShapeDtypeStruct((B, D_OPE, N_OPES), jnp.float32),
            jax.ShapeDtypeStruct((N_MAS, D_MA, B), jnp.float32),
        ],
    )(pt, ot, mt)

    pn = pl.pallas_call(
        functools.partial(_pass2_body, n_total),
        grid=(G2,),
        in_specs=[
            pl.BlockSpec((_BS2, N_MAS, N_OPES), lambda i: (i, 0, 0)),
            pl.BlockSpec((G1, 2, N_OPES), lambda i: (0, 0, 0)),
            pl.BlockSpec((32, 2, 16), lambda i: (0, 0, 0)),
        ],
        out_specs=pl.BlockSpec((_BS2, N_MAS, N_OPES), lambda i: (i, 0, 0)),
        out_shape=jax.ShapeDtypeStruct((B, N_MAS, N_OPES), jnp.float32),
    )(pt, parts, sc_parts)

    return (jnp.transpose(on, (0, 2, 1)),
            jnp.transpose(mn, (2, 0, 1)),
            jnp.transpose(pn, (0, 2, 1)))


# speedup vs baseline: 1.2537x; 1.2537x over previous
import functools

import jax
import jax.numpy as jnp
from jax import lax
from jax.experimental import pallas as pl
from jax.experimental.pallas import tpu as pltpu
from jax.experimental.pallas import tpu_sc as plsc

_BS1 = 12
_BS2 = 16
_SC_B = 64
_ROWS_PER_SAMPLE = 64
_LANES = 2048
_CH = 16


def _pass1_body(n_opes, n_mas,
                proc_ref, opes_ref, mas_ref,
                part_out, opes_out, mas_out):
    @pl.when(pl.program_id(0) == 0)
    def _mas():
        y = mas_ref[...]
        my = jnp.mean(y, axis=0, keepdims=True)
        dy = y - my
        vy = jnp.sum(dy * dy, axis=0, keepdims=True) * (1.0 / (n_mas - 1.0))
        mas_out[...] = dy / (jnp.sqrt(vy) + 1e-5)

    x = proc_ref[...] - 0.5
    ps = jnp.sum(x, axis=(0, 1))
    ps2 = jnp.sum(x * x, axis=(0, 1))
    part_out[...] = jnp.stack([ps, ps2]).reshape(1, 2, -1)

    z = opes_ref[...]
    m = jnp.mean(z, axis=2, keepdims=True)
    d = z - m
    v = jnp.sum(d * d, axis=2, keepdims=True) * (1.0 / (n_opes - 1.0))
    opes_out[...] = d / (jnp.sqrt(v) + 1e-5)


def _sc_reduce_body(rows_per_worker, start_row, x_hbm, out_hbm,
                    buf0, buf1, obuf, sem0, sem1):
    wid = lax.axis_index("s") * 2 + lax.axis_index("c")
    r0 = start_row + wid * rows_per_worker
    n_chunks = rows_per_worker // _CH
    bufs = (buf0, buf1)
    sems = (sem0, sem1)
    U = 8

    copies = [None] * n_chunks
    copies[0] = pltpu.async_copy(x_hbm.at[pl.ds(r0, _CH)], buf0, sem0)
    zeros = jnp.zeros((16,), jnp.float32)
    carry = (zeros,) * (2 * U)
    for c in range(n_chunks):
        if c + 1 < n_chunks:
            copies[c + 1] = pltpu.async_copy(
                x_hbm.at[pl.ds(r0 + (c + 1) * _CH, _CH)],
                bufs[(c + 1) % 2], sems[(c + 1) % 2])
        copies[c].wait()
        buf = bufs[c % 2]

        def _step(k, cr, buf=buf):
            row = k // 16
            base = (k % 16) * 128
            out = list(cr)
            for u in range(U):
                v = buf[row, pl.ds(base + u * 16, 16)]
                out[u] = out[u] + v
                out[U + u] = out[U + u] + v * v
            return tuple(out)

        carry = lax.fori_loop(0, _CH * 16, _step, carry)

    acc = carry[0]
    acc2 = carry[U]
    for u in range(1, U):
        acc = acc + carry[u]
        acc2 = acc2 + carry[U + u]
    obuf[0, :] = acc
    obuf[1, :] = acc2
    pltpu.sync_copy(obuf, out_hbm.at[wid])


def _pass2_body(n_total, n_sc, proc_ref, part_ref, scp_ref, proc_out):
    parts = part_ref[...]
    scp = scp_ref[...]
    s_u = jnp.sum(scp[:, 0:1, :])
    s2_u = jnp.sum(scp[:, 1:2, :])
    s = jnp.sum(parts[:, 0:1, :]) + (s_u - 0.5 * n_sc)
    s2 = jnp.sum(parts[:, 1:2, :]) + (s2_u - s_u + 0.25 * n_sc)
    n = float(n_total)
    gvar = (s2 - s * s / n) / (n - 1.0)
    ginv = 1.0 / (jnp.sqrt(gvar) + 1e-5)
    gmean = s / n
    proc_out[...] = ((proc_ref[...] - 0.5) - gmean) * ginv


def kernel(raw_opes, raw_mas, proc_time, batch_idxes, nums_opes):
    B, N_OPES, D_OPE = raw_opes.shape
    _, N_MAS, D_MA = raw_mas.shape
    n_total = B * N_OPES * N_MAS
    tc_b = B - _SC_B
    G1 = tc_b // _BS1
    G2 = B // _BS2
    bs_o = B // G1

    pt = jnp.transpose(proc_time, (0, 2, 1))
    ot = jnp.transpose(raw_opes, (0, 2, 1))
    mt = jnp.transpose(raw_mas, (1, 2, 0))
    pt2d = pt.reshape(B * N_MAS, N_OPES)

    rows_per_worker = _SC_B * _ROWS_PER_SAMPLE // 32
    start_row = tc_b * _ROWS_PER_SAMPLE

    sc_parts = pl.kernel(
        functools.partial(_sc_reduce_body, rows_per_worker, start_row),
        mesh=plsc.VectorSubcoreMesh(core_axis_name="c", subcore_axis_name="s"),
        out_type=jax.ShapeDtypeStruct((32, 2, 16), jnp.float32),
        scratch_types=[
            pltpu.VMEM((_CH, _LANES), jnp.float32),
            pltpu.VMEM((_CH, _LANES), jnp.float32),
            pltpu.VMEM((2, 16), jnp.float32),
            pltpu.SemaphoreType.DMA,
            pltpu.SemaphoreType.DMA,
        ],
        compiler_params=pltpu.CompilerParams(use_tc_tiling_on_sc=True),
    )(pt2d)

    parts, on, mn = pl.pallas_call(
        functools.partial(_pass1_body, N_OPES, N_MAS),
        grid=(G1,),
        in_specs=[
            pl.BlockSpec((_BS1, N_MAS, N_OPES), lambda i: (i, 0, 0)),
            pl.BlockSpec((bs_o, D_OPE, N_OPES), lambda i: (i, 0, 0)),
            pl.BlockSpec((N_MAS, D_MA, B), lambda i: (0, 0, 0)),
        ],
        out_specs=[
            pl.BlockSpec((1, 2, N_OPES), lambda i: (i, 0, 0)),
            pl.BlockSpec((bs_o, D_OPE, N_OPES), lambda i: (i, 0, 0)),
            pl.BlockSpec((N_MAS, D_MA, B), lambda i: (0, 0, 0)),
        ],
        out_shape=[
            jax.ShapeDtypeStruct((G1, 2, N_OPES), jnp.float32),
            jax.ShapeDtypeStruct((B, D_OPE, N_OPES), jnp.float32),
            jax.ShapeDtypeStruct((N_MAS, D_MA, B), jnp.float32),
        ],
    )(pt, ot, mt)

    n_sc = float(_SC_B * N_OPES * N_MAS)
    pn = pl.pallas_call(
        functools.partial(_pass2_body, n_total, n_sc),
        grid=(G2,),
        in_specs=[
            pl.BlockSpec((_BS2, N_MAS, N_OPES), lambda i: (i, 0, 0)),
            pl.BlockSpec((G1, 2, N_OPES), lambda i: (0, 0, 0)),
            pl.BlockSpec((32, 2, 16), lambda i: (0, 0, 0)),
        ],
        out_specs=pl.BlockSpec((_BS2, N_MAS, N_OPES), lambda i: (i, 0, 0)),
        out_shape=jax.ShapeDtypeStruct((B, N_MAS, N_OPES), jnp.float32),
    )(pt, parts, sc_parts)

    return (jnp.transpose(on, (0, 2, 1)),
            jnp.transpose(mn, (2, 0, 1)),
            jnp.transpose(pn, (0, 2, 1)))


# speedup vs baseline: 1.3058x; 1.0415x over previous
import functools

import jax
import jax.numpy as jnp
from jax import lax
from jax.experimental import pallas as pl
from jax.experimental.pallas import tpu as pltpu
from jax.experimental.pallas import tpu_sc as plsc

_BS1 = 13
_BS2 = 16
_SC_B = 48
_ROWS_PER_SAMPLE = 64
_LANES = 2048
_CH = 16


def _pass1_body(n_opes, n_mas,
                proc_ref, opes_ref, mas_ref,
                part_out, opes_out, mas_out):
    @pl.when(pl.program_id(0) == 0)
    def _mas():
        y = mas_ref[...]
        my = jnp.mean(y, axis=0, keepdims=True)
        dy = y - my
        vy = jnp.sum(dy * dy, axis=0, keepdims=True) * (1.0 / (n_mas - 1.0))
        mas_out[...] = dy / (jnp.sqrt(vy) + 1e-5)

    x = proc_ref[...] - 0.5
    ps = jnp.sum(x, axis=(0, 1))
    ps2 = jnp.sum(x * x, axis=(0, 1))
    part_out[...] = jnp.stack([ps, ps2]).reshape(1, 2, -1)

    z = opes_ref[...]
    m = jnp.mean(z, axis=2, keepdims=True)
    d = z - m
    v = jnp.sum(d * d, axis=2, keepdims=True) * (1.0 / (n_opes - 1.0))
    opes_out[...] = d / (jnp.sqrt(v) + 1e-5)


def _sc_reduce_body(rows_per_worker, start_row, x_hbm, out_hbm,
                    buf0, buf1, obuf, sem0, sem1):
    wid = lax.axis_index("s") * 2 + lax.axis_index("c")
    r0 = start_row + wid * rows_per_worker
    n_chunks = rows_per_worker // _CH
    bufs = (buf0, buf1)
    sems = (sem0, sem1)
    U = 16

    copies = [None] * n_chunks
    copies[0] = pltpu.async_copy(x_hbm.at[pl.ds(r0, _CH)], buf0, sem0)
    zeros = jnp.zeros((16,), jnp.float32)
    carry = (zeros,) * (2 * U)
    for c in range(n_chunks):
        if c + 1 < n_chunks:
            copies[c + 1] = pltpu.async_copy(
                x_hbm.at[pl.ds(r0 + (c + 1) * _CH, _CH)],
                bufs[(c + 1) % 2], sems[(c + 1) % 2])
        copies[c].wait()
        buf = bufs[c % 2]

        def _step(k, cr, buf=buf):
            row = k // 8
            base = (k % 8) * 256
            out = list(cr)
            for u in range(U):
                v = buf[row, pl.ds(base + u * 16, 16)]
                out[u] = out[u] + v
                out[U + u] = out[U + u] + v * v
            return tuple(out)

        carry = lax.fori_loop(0, _CH * 8, _step, carry)

    acc = carry[0]
    acc2 = carry[U]
    for u in range(1, U):
        acc = acc + carry[u]
        acc2 = acc2 + carry[U + u]
    obuf[0, :] = acc
    obuf[1, :] = acc2
    pltpu.sync_copy(obuf, out_hbm.at[wid])


def _pass2_body(n_total, n_sc, proc_ref, part_ref, scp_ref, proc_out):
    parts = part_ref[...]
    scp = scp_ref[...]
    s_u = jnp.sum(scp[:, 0:1, :])
    s2_u = jnp.sum(scp[:, 1:2, :])
    s = jnp.sum(parts[:, 0:1, :]) + (s_u - 0.5 * n_sc)
    s2 = jnp.sum(parts[:, 1:2, :]) + (s2_u - s_u + 0.25 * n_sc)
    n = float(n_total)
    gvar = (s2 - s * s / n) / (n - 1.0)
    ginv = 1.0 / (jnp.sqrt(gvar) + 1e-5)
    gmean = s / n
    proc_out[...] = ((proc_ref[...] - 0.5) - gmean) * ginv


def kernel(raw_opes, raw_mas, proc_time, batch_idxes, nums_opes):
    B, N_OPES, D_OPE = raw_opes.shape
    _, N_MAS, D_MA = raw_mas.shape
    n_total = B * N_OPES * N_MAS
    tc_b = B - _SC_B
    G1 = tc_b // _BS1
    G2 = B // _BS2
    bs_o = B // G1

    pt = jnp.transpose(proc_time, (0, 2, 1))
    ot = jnp.transpose(raw_opes, (0, 2, 1))
    mt = jnp.transpose(raw_mas, (1, 2, 0))
    pt2d = pt.reshape(B * N_MAS, N_OPES)

    rows_per_worker = _SC_B * _ROWS_PER_SAMPLE // 32
    start_row = tc_b * _ROWS_PER_SAMPLE

    sc_parts = pl.kernel(
        functools.partial(_sc_reduce_body, rows_per_worker, start_row),
        mesh=plsc.VectorSubcoreMesh(core_axis_name="c", subcore_axis_name="s"),
        out_type=jax.ShapeDtypeStruct((32, 2, 16), jnp.float32),
        scratch_types=[
            pltpu.VMEM((_CH, _LANES), jnp.float32),
            pltpu.VMEM((_CH, _LANES), jnp.float32),
            pltpu.VMEM((2, 16), jnp.float32),
            pltpu.SemaphoreType.DMA,
            pltpu.SemaphoreType.DMA,
        ],
        compiler_params=pltpu.CompilerParams(use_tc_tiling_on_sc=True),
    )(pt2d)

    parts, on, mn = pl.pallas_call(
        functools.partial(_pass1_body, N_OPES, N_MAS),
        grid=(G1,),
        in_specs=[
            pl.BlockSpec((_BS1, N_MAS, N_OPES), lambda i: (i, 0, 0)),
            pl.BlockSpec((bs_o, D_OPE, N_OPES), lambda i: (i, 0, 0)),
            pl.BlockSpec((N_MAS, D_MA, B), lambda i: (0, 0, 0)),
        ],
        out_specs=[
            pl.BlockSpec((1, 2, N_OPES), lambda i: (i, 0, 0)),
            pl.BlockSpec((bs_o, D_OPE, N_OPES), lambda i: (i, 0, 0)),
            pl.BlockSpec((N_MAS, D_MA, B), lambda i: (0, 0, 0)),
        ],
        out_shape=[
            jax.ShapeDtypeStruct((G1, 2, N_OPES), jnp.float32),
            jax.ShapeDtypeStruct((B, D_OPE, N_OPES), jnp.float32),
            jax.ShapeDtypeStruct((N_MAS, D_MA, B), jnp.float32),
        ],
    )(pt, ot, mt)

    n_sc = float(_SC_B * N_OPES * N_MAS)
    pn = pl.pallas_call(
        functools.partial(_pass2_body, n_total, n_sc),
        grid=(G2,),
        in_specs=[
            pl.BlockSpec((_BS2, N_MAS, N_OPES), lambda i: (i, 0, 0)),
            pl.BlockSpec((G1, 2, N_OPES), lambda i: (0, 0, 0)),
            pl.BlockSpec((32, 2, 16), lambda i: (0, 0, 0)),
        ],
        out_specs=pl.BlockSpec((_BS2, N_MAS, N_OPES), lambda i: (i, 0, 0)),
        out_shape=jax.ShapeDtypeStruct((B, N_MAS, N_OPES), jnp.float32),
    )(pt, parts, sc_parts)

    return (jnp.transpose(on, (0, 2, 1)),
            jnp.transpose(mn, (2, 0, 1)),
            jnp.transpose(pn, (0, 2, 1)))


# speedup vs baseline: 1.4245x; 1.0909x over previous
import functools

import jax
import jax.numpy as jnp
from jax.experimental import pallas as pl
from jax.experimental.pallas import tpu as pltpu

_BS = 16


def _body(g, n_total, n_opes, n_mas,
          proc_ref, opes_ref, mas_ref,
          proc_out, opes_out, mas_out, acc_ref):
    i = pl.program_id(0)

    @pl.when(i == 0)
    def _init():
        acc_ref[...] = jnp.zeros_like(acc_ref)
        y = mas_ref[...]
        my = jnp.mean(y, axis=0, keepdims=True)
        dy = y - my
        vy = jnp.sum(dy * dy, axis=0, keepdims=True) * (1.0 / (n_mas - 1.0))
        mas_out[...] = dy / (jnp.sqrt(vy) + 1e-5)

    @pl.when(i < g)
    def _phase0():
        x = proc_ref[...] - 0.5
        ps = jnp.sum(x, axis=(0, 1))
        ps2 = jnp.sum(x * x, axis=(0, 1))
        acc_ref[...] += jnp.stack([ps, ps2])

        z = opes_ref[...]
        m = jnp.mean(z, axis=2, keepdims=True)
        d = z - m
        v = jnp.sum(d * d, axis=2, keepdims=True) * (1.0 / (n_opes - 1.0))
        opes_out[...] = d / (jnp.sqrt(v) + 1e-5)

    @pl.when(i >= g)
    def _phase1():
        acc = acc_ref[...]
        s = jnp.sum(acc[0:1, :])
        s2 = jnp.sum(acc[1:2, :])
        n = float(n_total)
        gvar = (s2 - s * s / n) / (n - 1.0)
        ginv = 1.0 / (jnp.sqrt(gvar) + 1e-5)
        gmean = s / n
        proc_out[...] = ((proc_ref[...] - 0.5) - gmean) * ginv


def kernel(raw_opes, raw_mas, proc_time, batch_idxes, nums_opes):
    B, N_OPES, D_OPE = raw_opes.shape
    _, N_MAS, D_MA = raw_mas.shape
    n_total = B * N_OPES * N_MAS
    G = B // _BS
    bs = B // G

    pt = jnp.transpose(proc_time, (0, 2, 1))
    ot = jnp.transpose(raw_opes, (0, 2, 1))
    mt = jnp.transpose(raw_mas, (1, 2, 0))

    pn, on, mn = pl.pallas_call(
        functools.partial(_body, G, n_total, N_OPES, N_MAS),
        grid=(2 * G,),
        in_specs=[
            pl.BlockSpec((_BS, N_MAS, N_OPES), lambda i: (i % G, 0, 0)),
            pl.BlockSpec((bs, D_OPE, N_OPES),
                         lambda i: (jnp.minimum(i, G - 1), 0, 0)),
            pl.BlockSpec((N_MAS, D_MA, B), lambda i: (0, 0, 0)),
        ],
        out_specs=[
            pl.BlockSpec((_BS, N_MAS, N_OPES),
                         lambda i: (jnp.maximum(i - G, 0), 0, 0)),
            pl.BlockSpec((bs, D_OPE, N_OPES),
                         lambda i: (jnp.minimum(i, G - 1), 0, 0)),
            pl.BlockSpec((N_MAS, D_MA, B), lambda i: (0, 0, 0)),
        ],
        out_shape=[
            jax.ShapeDtypeStruct((B, N_MAS, N_OPES), jnp.float32),
            jax.ShapeDtypeStruct((B, D_OPE, N_OPES), jnp.float32),
            jax.ShapeDtypeStruct((N_MAS, D_MA, B), jnp.float32),
        ],
        scratch_shapes=[pltpu.VMEM((2, N_OPES), jnp.float32)],
    )(pt, ot, mt)

    return (jnp.transpose(on, (0, 2, 1)),
            jnp.transpose(mn, (2, 0, 1)),
            jnp.transpose(pn, (0, 2, 1)))
